# Initial kernel scaffold; baseline (speedup 1.0000x reference)
import jax
import jax.numpy as jnp
from jax.experimental import pallas as pl


def kernel(x, edge_index, W1, b1, W2, b2):
    raise NotImplementedError("write your pallas kernel here")


# speedup vs baseline: 2.6712x; 2.6712x over previous
import functools

import jax
import jax.numpy as jnp
from jax import lax
from jax.experimental import pallas as pl
from jax.experimental.pallas import tpu as pltpu
from jax.experimental.pallas import tpu_sc as plsc


def _padded_nodes(n_nodes: int) -> int:
    return -(-n_nodes // 128) * 128


def _make_seg_sum(n_nodes: int, n_edges: int, half: int):
    NSC = 2
    NT = 16
    CHUNK = 80
    ept = n_edges // NT
    n_chunks = ept // CHUNK
    assert ept * NT == n_edges and n_chunks * CHUNK == ept
    n_pad = _padded_nodes(n_nodes)
    rpt = n_pad // NT

    mesh = plsc.VectorSubcoreMesh(core_axis_name="c", subcore_axis_name="s")

    @functools.partial(
        pl.kernel,
        out_type=jax.ShapeDtypeStruct((NSC * n_pad, half), jnp.float32),
        mesh=mesh,
        compiler_params=pltpu.CompilerParams(use_tc_tiling_on_sc=False),
        scratch_types=[
            pltpu.VMEM((CHUNK,), jnp.int32),
            pltpu.VMEM((CHUNK,), jnp.int32),
            pltpu.VMEM((CHUNK, half), jnp.float32),
            pltpu.VMEM_SHARED((n_pad, half), jnp.float32),
            pltpu.SemaphoreType.DMA,
        ],
    )
    def seg_sum(xs_hbm, src_hbm, dst_hbm, zeros_hbm, out_hbm,
                srcv, dstv, rows, acc, sem):
        c = lax.axis_index("c")
        s = lax.axis_index("s")

        pltpu.sync_copy(zeros_hbm, acc.at[pl.ds(s * rpt, rpt)])
        plsc.subcore_barrier()

        tbl_off = c * n_nodes

        def body(j, carry):
            e0 = s * ept + j * CHUNK
            pltpu.sync_copy(src_hbm.at[pl.ds(e0, CHUNK)], srcv)
            pltpu.sync_copy(dst_hbm.at[pl.ds(e0, CHUNK)], dstv)
            for i in range(CHUNK // 16):
                srcv[pl.ds(i * 16, 16)] = srcv[pl.ds(i * 16, 16)] + tbl_off
            pltpu.async_copy(xs_hbm.at[srcv], rows, sem).wait()
            pltpu.sync_copy(rows, acc.at[dstv], add=True)
            return carry

        lax.fori_loop(0, n_chunks, body, 0)
        plsc.subcore_barrier()

        pltpu.sync_copy(acc.at[pl.ds(s * rpt, rpt)],
                        out_hbm.at[pl.ds(c * n_pad + s * rpt, rpt)])

    return seg_sum


def _mlp_call(x, a0, a1, W1, b1, W2, b2, *, final_relu: bool,
              emit_split: bool, block_rows: int):
    n, d = x.shape
    half = d // 2
    nb = n // block_rows
    assert nb * block_rows == n

    def body(x_ref, a0_ref, a1_ref, w1_ref, b1_ref, w2_ref, b2_ref, *outs):
        u = x_ref[...] + jnp.concatenate([a0_ref[...], a1_ref[...]], axis=1)
        t = jnp.maximum(jnp.dot(u, w1_ref[...],
                                preferred_element_type=jnp.float32)
                        + b1_ref[...], 0.0)
        o = jnp.dot(t, w2_ref[...], preferred_element_type=jnp.float32) \
            + b2_ref[...]
        if final_relu:
            o = jnp.maximum(o, 0.0)
        outs[0][...] = o
        if emit_split:
            outs[1][0] = o[:, :half]
            outs[1][1] = o[:, half:]

    out_shapes = [jax.ShapeDtypeStruct((n, d), jnp.float32)]
    out_specs = [pl.BlockSpec((block_rows, d), lambda i: (i, 0))]
    if emit_split:
        out_shapes.append(jax.ShapeDtypeStruct((2, n, half), jnp.float32))
        out_specs.append(pl.BlockSpec((2, block_rows, half),
                                      lambda i: (0, i, 0)))

    return pl.pallas_call(
        body,
        grid=(nb,),
        in_specs=[
            pl.BlockSpec((block_rows, d), lambda i: (i, 0)),
            pl.BlockSpec((block_rows, half), lambda i: (i, 0)),
            pl.BlockSpec((block_rows, half), lambda i: (i, 0)),
            pl.BlockSpec((d, d), lambda i: (0, 0)),
            pl.BlockSpec((1, d), lambda i: (0, 0)),
            pl.BlockSpec((d, d), lambda i: (0, 0)),
            pl.BlockSpec((1, d), lambda i: (0, 0)),
        ],
        out_specs=out_specs,
        out_shape=out_shapes,
    )(x, a0, a1, W1, b1.reshape(1, d), W2, b2.reshape(1, d))


def kernel(x, edge_index, W1, b1, W2, b2):
    n, d = x.shape
    e = edge_index.shape[1]
    half = d // 2
    src = edge_index[0]
    dst = edge_index[1]

    xs = x.reshape(n, 2, half).transpose(1, 0, 2).reshape(2 * n, half)
    n_pad = _padded_nodes(n)
    zeros = jnp.zeros((n_pad // 16, half), jnp.float32)

    seg_sum = _make_seg_sum(n, e, half)

    agg1 = seg_sum(xs, src, dst, zeros)
    h, hs = _mlp_call(x, agg1[:n], agg1[n_pad:n_pad + n], W1, b1, W2, b2,
                      final_relu=True, emit_split=True, block_rows=2000)
    agg2 = seg_sum(hs.reshape(2 * n, half), src, dst, zeros)
    (z,) = _mlp_call(h, agg2[:n], agg2[n_pad:n_pad + n], W1, b1, W2, b2,
                     final_relu=False, emit_split=False, block_rows=2000)
    return z
